# baseline (speedup 1.0000x reference)
import functools

import jax
import jax.numpy as jnp
from jax import lax
from jax.experimental import pallas as pl
from jax.experimental.pallas import tpu as pltpu

B = 4096
L = 196
D = 64
MAX_T = 160
TBL_PAD = 256

BB = 64


def _body(ts_ref, patch_ref, speed_ref, fcw_ref, fcb_ref, table_ref, out_ref):
    token = speed_ref[...] * fcw_ref[...] + fcb_ref[...]
    ts = ts_ref[...]
    iota = lax.broadcasted_iota(jnp.int32, (BB, TBL_PAD), 1)
    eq = iota == ts
    temb = jnp.sum(jnp.where(eq, table_ref[...], 0.0), axis=1)
    out_ref[:, :L, :D] = patch_ref[...]
    out_ref[:, L:L + 1, :D] = token[:, None, :]
    out_ref[:, :, D:D + 1] = jnp.broadcast_to(
        temb[:, None, None], (BB, L + 1, 1))


@jax.jit
def kernel(patch, speed, time_step, fc_w, fc_b, time_table):
    fcw_row = fc_w.reshape(1, D)
    fcb_row = fc_b.reshape(1, D)
    table_row = jnp.pad(time_table.reshape(-1), (0, TBL_PAD - (MAX_T + 1)))
    table_row = table_row.reshape(1, TBL_PAD)
    ts = time_step.astype(jnp.int32).reshape(B, 1)

    grid = (B // BB,)
    out = pl.pallas_call(
        _body,
        grid=grid,
        in_specs=[
            pl.BlockSpec((BB, 1), lambda i: (i, 0)),
            pl.BlockSpec((BB, L, D), lambda i: (i, 0, 0)),
            pl.BlockSpec((BB, 1), lambda i: (i, 0)),
            pl.BlockSpec((1, D), lambda i: (0, 0)),
            pl.BlockSpec((1, D), lambda i: (0, 0)),
            pl.BlockSpec((1, TBL_PAD), lambda i: (0, 0)),
        ],
        out_specs=pl.BlockSpec((BB, L + 1, D + 1), lambda i: (i, 0, 0)),
        out_shape=jax.ShapeDtypeStruct((B, L + 1, D + 1), jnp.float32),
    )(ts, patch, speed, fcw_row, fcb_row, table_row)
    return out


# speedup vs baseline: 1.0084x; 1.0084x over previous
import jax
import jax.numpy as jnp
from jax import lax
from jax.experimental import pallas as pl
from jax.experimental.pallas import tpu as pltpu

B = 4096
L = 196
D = 64
MAX_T = 160
TBL_PAD = 256

CB = 32
RING = 4
NCH = B // CB
GROUPS = NCH // RING


def _in_copy(patch_hbm, in_bufs, in_sems, r, c):
    return pltpu.make_async_copy(
        patch_hbm.at[pl.ds(c * CB, CB)], in_bufs.at[r], in_sems.at[r])


def _out_copy(out_hbm, out_bufs, out_sems, r, c):
    return pltpu.make_async_copy(
        out_bufs.at[r], out_hbm.at[pl.ds(c * CB, CB)], out_sems.at[r])


def _body(ts_ref, speed_ref, fcw_ref, fcb_ref, table_ref,
          patch_hbm, out_hbm, in_bufs, out_bufs, in_sems, out_sems):
    for r in range(RING):
        _in_copy(patch_hbm, in_bufs, in_sems, r, r).start()

    def group(g, _):
        for r in range(RING):
            c = g * RING + r
            off = pl.ds(c * CB, CB)
            _in_copy(patch_hbm, in_bufs, in_sems, r, c).wait()

            @pl.when(g > 0)
            def _():
                _out_copy(out_hbm, out_bufs, out_sems, r, c).wait()

            token = speed_ref[off, :] * fcw_ref[...] + fcb_ref[...]
            ts = ts_ref[off, :]
            iota = lax.broadcasted_iota(jnp.int32, (CB, TBL_PAD), 1)
            temb = jnp.sum(jnp.where(iota == ts, table_ref[...], 0.0),
                           axis=1, keepdims=True)
            out_bufs[r, :, :L, :D] = in_bufs[r]
            out_bufs[r, :, L:L + 1, :D] = token[:, None, :]
            out_bufs[r, :, :, D:D + 1] = jnp.broadcast_to(
                temb[:, None, :], (CB, L + 1, 1))
            _out_copy(out_hbm, out_bufs, out_sems, r, c).start()

            @pl.when(g < GROUPS - 1)
            def _():
                _in_copy(patch_hbm, in_bufs, in_sems, r, c + RING).start()
        return 0

    lax.fori_loop(0, GROUPS, group, 0)
    for r in range(RING):
        _out_copy(out_hbm, out_bufs, out_sems, r, NCH - RING + r).wait()


@jax.jit
def kernel(patch, speed, time_step, fc_w, fc_b, time_table):
    fcw_row = fc_w.reshape(1, D)
    fcb_row = fc_b.reshape(1, D)
    table_row = jnp.pad(time_table.reshape(-1), (0, TBL_PAD - (MAX_T + 1)))
    table_row = table_row.reshape(1, TBL_PAD)
    ts = time_step.astype(jnp.int32).reshape(B, 1)

    out = pl.pallas_call(
        _body,
        in_specs=[
            pl.BlockSpec(memory_space=pltpu.MemorySpace.VMEM),
            pl.BlockSpec(memory_space=pltpu.MemorySpace.VMEM),
            pl.BlockSpec(memory_space=pltpu.MemorySpace.VMEM),
            pl.BlockSpec(memory_space=pltpu.MemorySpace.VMEM),
            pl.BlockSpec(memory_space=pltpu.MemorySpace.VMEM),
            pl.BlockSpec(memory_space=pltpu.MemorySpace.HBM),
        ],
        out_specs=pl.BlockSpec(memory_space=pltpu.MemorySpace.HBM),
        out_shape=jax.ShapeDtypeStruct((B, L + 1, D + 1), jnp.float32),
        scratch_shapes=[
            pltpu.VMEM((RING, CB, L, D), jnp.float32),
            pltpu.VMEM((RING, CB, L + 1, D + 1), jnp.float32),
            pltpu.SemaphoreType.DMA((RING,)),
            pltpu.SemaphoreType.DMA((RING,)),
        ],
    )(ts, speed, fcw_row, fcb_row, table_row, patch)
    return out


# speedup vs baseline: 6.5728x; 6.5183x over previous
import jax
import jax.numpy as jnp
from jax import lax
from jax.experimental import pallas as pl
from jax.experimental.pallas import tpu as pltpu

B = 4096
L = 196
D = 64
MAX_T = 160
TBL = 168
BC = 128


def _body(ts_ref, speed_ref, fcw_ref, fcb_ref, table_ref, patch_ref, out_ref):
    x = patch_ref[...]
    out_ref[:D, :L, :] = jnp.transpose(x, (1, 0, 2))
    token = speed_ref[...] * fcw_ref[...] + fcb_ref[...]
    out_ref[:D, L, :] = token
    ts = ts_ref[...]
    acc = jnp.zeros((1, BC), jnp.float32)
    for k in range(0, TBL, 8):
        i8 = lax.broadcasted_iota(jnp.int32, (8, BC), 0) + k
        vals = jnp.where(i8 == ts, table_ref[k:k + 8, :], 0.0)
        acc = acc + jnp.sum(vals, axis=0, keepdims=True)
    out_ref[D, :, :] = jnp.broadcast_to(acc, (L + 1, BC))


@jax.jit
def kernel(patch, speed, time_step, fc_w, fc_b, time_table):
    patch_t = jnp.transpose(patch, (1, 2, 0))
    speed_row = speed.reshape(1, B)
    ts_row = time_step.astype(jnp.int32).reshape(1, B)
    fcw_col = fc_w.reshape(D, 1)
    fcb_col = fc_b.reshape(D, 1)
    table_col = jnp.pad(time_table, ((0, TBL - (MAX_T + 1)), (0, 0)))

    grid = (B // BC,)
    out_t = pl.pallas_call(
        _body,
        grid=grid,
        in_specs=[
            pl.BlockSpec((1, BC), lambda i: (0, i)),
            pl.BlockSpec((1, BC), lambda i: (0, i)),
            pl.BlockSpec((D, 1), lambda i: (0, 0)),
            pl.BlockSpec((D, 1), lambda i: (0, 0)),
            pl.BlockSpec((TBL, 1), lambda i: (0, 0)),
            pl.BlockSpec((L, D, BC), lambda i: (0, 0, i)),
        ],
        out_specs=pl.BlockSpec((D + 1, L + 1, BC), lambda i: (0, 0, i)),
        out_shape=jax.ShapeDtypeStruct((D + 1, L + 1, B), jnp.float32),
    )(ts_row, speed_row, fcw_col, fcb_col, table_col, patch_t)
    return jnp.transpose(out_t, (2, 1, 0))
